# Initial kernel scaffold; baseline (speedup 1.0000x reference)
import jax
import jax.numpy as jnp
from jax.experimental import pallas as pl


def kernel(red, blue, strengths):
    raise NotImplementedError("write your pallas kernel here")


# speedup vs baseline: 2.6070x; 2.6070x over previous
import functools

import jax
import jax.numpy as jnp
from jax import lax
from jax.experimental import pallas as pl
from jax.experimental.pallas import tpu as pltpu, tpu_sc as plsc

NUM_CHAMPIONS = 100000
BATCH = 16384
TEAM = 5
NUM_WORKERS = 32
ROWS_PER_WORKER = BATCH // NUM_WORKERS
LANES = 16
CHUNKS = ROWS_PER_WORKER // LANES


@functools.partial(
    pl.kernel,
    mesh=plsc.VectorSubcoreMesh(core_axis_name="c", subcore_axis_name="s"),
    out_type=jax.ShapeDtypeStruct((BATCH,), jnp.float32),
    compiler_params=pltpu.CompilerParams(needs_layout_passes=False),
    scratch_types=[
        pltpu.VMEM((NUM_CHAMPIONS,), jnp.float32),
        pltpu.VMEM((2 * TEAM, ROWS_PER_WORKER), jnp.int32),
        pltpu.VMEM((ROWS_PER_WORKER,), jnp.float32),
    ],
)
def _pam_kernel(table_hbm, idx_hbm, out_hbm, table_v, idx_v, out_v):
    wid = lax.axis_index("s") * 2 + lax.axis_index("c")
    base = wid * ROWS_PER_WORKER

    pltpu.sync_copy(table_hbm, table_v)
    pltpu.sync_copy(idx_hbm.at[wid], idx_v)

    for i in range(CHUNKS):
        sl = pl.ds(i * LANES, LANES)
        acc = plsc.load_gather(table_v, [idx_v[0, sl]])
        for t in range(1, TEAM):
            acc = acc + plsc.load_gather(table_v, [idx_v[t, sl]])
        for t in range(TEAM, 2 * TEAM):
            acc = acc - plsc.load_gather(table_v, [idx_v[t, sl]])
        out_v[sl] = 1.0 / (1.0 + jnp.exp(-acc))

    pltpu.sync_copy(out_v, out_hbm.at[pl.ds(base, ROWS_PER_WORKER)])


def kernel(red, blue, strengths):
    idx = jnp.concatenate([red.T, blue.T], axis=0).astype(jnp.int32)
    idx = idx.reshape(2 * TEAM, NUM_WORKERS, ROWS_PER_WORKER).transpose(1, 0, 2)
    return _pam_kernel(strengths, idx)
